# baseline (speedup 1.0000x reference)
import functools

import jax
import jax.numpy as jnp
from jax import lax
from jax.experimental import pallas as pl
from jax.experimental.pallas import tpu as pltpu
from jax.experimental.pallas import tpu_sc as plsc

_B, _C, _N = 8, 17, 2048
_K = 7
_RB = 512
_NRB = _N // _RB
_D_T = 289
_D_PAD = 384
_NV = _D_PAD // 16

_NW = 32
_RPW = (_B * _N) // _NW
_CH = 8
_NCH = _RPW // _CH
_KP = _K + 1


def _dot3(a, b, dims):
    a_hi = a.astype(jnp.bfloat16).astype(jnp.float32)
    a_lo = a - a_hi
    b_hi = b.astype(jnp.bfloat16).astype(jnp.float32)
    b_lo = b - b_hi
    dd = (dims, ((), ()))
    out = lax.dot_general(a_hi, b_hi, dd, preferred_element_type=jnp.float32)
    out += lax.dot_general(a_hi, b_lo, dd, preferred_element_type=jnp.float32)
    out += lax.dot_general(a_lo, b_hi, dd, preferred_element_type=jnp.float32)
    return out


def _topk_kernel(x_rows_ref, x_full_ref, lblr_ref, lblc_ref, idx_ref, w_ref):
    b = pl.program_id(0)

    xr = x_rows_ref[0]
    xf = x_full_ref[0]
    ones_c = jnp.ones((_C, 1), dtype=jnp.float32)

    s_col = lax.dot_general(xr * xr, ones_c, (((0,), (0,)), ((), ())),
                            preferred_element_type=jnp.float32)
    s_row = lax.dot_general(ones_c, xf * xf, (((0,), (0,)), ((), ())),
                            preferred_element_type=jnp.float32)
    g = _dot3(xr, xf, ((0,), (0,)))
    d = s_col + s_row - 2.0 * g

    lbl_row = lblr_ref[0]
    lbl_col = lblc_ref[0]

    iota = lax.broadcasted_iota(jnp.int32, (_RB, _N), 1)
    bits = lax.bitcast_convert_type(jnp.maximum(d, 0.0), jnp.int32)
    key = (bits & jnp.int32(-2048)) | iota

    for t in range(_KP):
        mkey = jnp.min(key, axis=1, keepdims=True)
        cmp = key == mkey
        dsel = jnp.min(jnp.where(cmp, d, 3.0e38), axis=1,
                       keepdims=True)
        lbl_j = jnp.max(jnp.where(cmp, lbl_row, -1), axis=1,
                        keepdims=True)
        key = jnp.where(cmp, jnp.int32(0x7FFFFFFF), key)
        sign = jnp.where(lbl_j == lbl_col, 1.0, -1.0).astype(jnp.float32)
        idx_ref[:, t:t + 1] = (mkey & jnp.int32(2047)) + b * _N
        w_ref[:, t:t + 1] = sign * jnp.exp(-0.5 * dsel)


def _pair_loss_kernel(t_hbm, idx_hbm, w_hbm, out_hbm,
                      idx_v, w_v, tj_v, ti_v, acc_v, sem):
    info = plsc.get_sparse_core_info()
    nc = info.num_cores
    wid = lax.axis_index("s") * nc + lax.axis_index("c")

    def chunk_body(c, acc):
        base_r = wid * _RPW + c * _CH
        base_p = base_r * _KP
        pltpu.sync_copy(idx_hbm.at[pl.ds(base_p, _CH * _KP)], idx_v)
        pltpu.sync_copy(w_hbm.at[pl.ds(base_p, _CH * _KP)], w_v)
        pltpu.sync_copy(t_hbm.at[pl.ds(base_r, _CH)], ti_v)
        pltpu.async_copy(t_hbm.at[idx_v], tj_v, sem).wait()

        for grp in range(_CH * _KP // 16):
            w16 = w_v[pl.ds(grp * 16, 16)]
            for r2 in range(2):
                row = grp * 2 + r2
                ti = [ti_v[row, pl.ds(j * 16, 16)] for j in range(_NV)]
                for k in range(_KP):
                    pair = row * _KP + k
                    pacc = jnp.zeros((16,), jnp.float32)
                    for j in range(_NV):
                        diff = ti[j] - tj_v[pair, pl.ds(j * 16, 16)]
                        pacc += diff * diff
                    acc += w16[r2 * _KP + k] * pacc
        return acc

    acc = lax.fori_loop(0, _NCH, chunk_body, jnp.zeros((16,), jnp.float32))
    acc_v[...] = acc
    pltpu.sync_copy(acc_v, out_hbm.at[wid])


def kernel(logits, labels, ins_T):
    x = logits

    idx, wsel = pl.pallas_call(
        _topk_kernel,
        grid=(_B, _NRB),
        in_specs=[
            pl.BlockSpec((1, _C, _RB), lambda b, rb: (b, 0, rb)),
            pl.BlockSpec((1, _C, _N), lambda b, rb: (b, 0, 0)),
            pl.BlockSpec((1, 1, _N), lambda b, rb: (b, 0, 0)),
            pl.BlockSpec((1, _RB, 1), lambda b, rb: (b, rb, 0)),
        ],
        out_specs=[
            pl.BlockSpec((_RB, _KP), lambda b, rb: (b * _NRB + rb, 0)),
            pl.BlockSpec((_RB, _KP), lambda b, rb: (b * _NRB + rb, 0)),
        ],
        out_shape=[
            jax.ShapeDtypeStruct((_B * _N, _KP), jnp.int32),
            jax.ShapeDtypeStruct((_B * _N, _KP), jnp.float32),
        ],
    )(x, x, labels.reshape(_B, 1, _N), labels.reshape(_B, _N, 1))

    t_pad = jnp.pad(ins_T.reshape(_B * _N, _D_T),
                    ((0, 0), (0, _D_PAD - _D_T)))
    mesh = plsc.VectorSubcoreMesh(core_axis_name="c", subcore_axis_name="s")

    sc = functools.partial(
        pl.kernel, mesh=mesh,
        out_type=jax.ShapeDtypeStruct((_NW, 16), jnp.float32),
        scratch_types=[
            pltpu.VMEM((_CH * _KP,), jnp.int32),
            pltpu.VMEM((_CH * _KP,), jnp.float32),
            pltpu.VMEM((_CH * _KP, _D_PAD), jnp.float32),
            pltpu.VMEM((_CH, _D_PAD), jnp.float32),
            pltpu.VMEM((16,), jnp.float32),
            pltpu.SemaphoreType.DMA,
        ],
    )(_pair_loss_kernel)

    partials = sc(t_pad, idx.reshape(-1), wsel.reshape(-1))

    return jnp.sum(partials) / jnp.float32(_B * _N * _K)


# speedup vs baseline: 1.0243x; 1.0243x over previous
import functools

import jax
import jax.numpy as jnp
from jax import lax
from jax.experimental import pallas as pl
from jax.experimental.pallas import tpu as pltpu
from jax.experimental.pallas import tpu_sc as plsc

_B, _C, _N = 8, 17, 2048
_K = 7
_RB = 512
_NRB = _N // _RB
_D_T = 289
_D_PAD = 384
_NV = _D_PAD // 16

_NW = 32
_RPW = (_B * _N) // _NW
_CH = 8
_NCH = _RPW // _CH


def _dot3(a, b, dims):
    a_hi = a.astype(jnp.bfloat16).astype(jnp.float32)
    a_lo = a - a_hi
    b_hi = b.astype(jnp.bfloat16).astype(jnp.float32)
    b_lo = b - b_hi
    dd = (dims, ((), ()))
    out = lax.dot_general(a_hi, b_hi, dd, preferred_element_type=jnp.float32)
    out += lax.dot_general(a_hi, b_lo, dd, preferred_element_type=jnp.float32)
    out += lax.dot_general(a_lo, b_hi, dd, preferred_element_type=jnp.float32)
    return out


def _topk_kernel(x_rows_ref, x_full_ref, lblr_ref, lblc_ref, idx_ref, w_ref):
    b = pl.program_id(0)
    rb = pl.program_id(1)

    xr = x_rows_ref[0]
    xf = x_full_ref[0]
    ones_c = jnp.ones((_C, 1), dtype=jnp.float32)

    s_col = lax.dot_general(xr * xr, ones_c, (((0,), (0,)), ((), ())),
                            preferred_element_type=jnp.float32)
    s_row = lax.dot_general(ones_c, xf * xf, (((0,), (0,)), ((), ())),
                            preferred_element_type=jnp.float32)
    g = _dot3(xr, xf, ((0,), (0,)))
    d = s_col + s_row - 2.0 * g

    lbl_row = lblr_ref[0]
    lbl_col = lblc_ref[0]

    iota = lax.broadcasted_iota(jnp.int32, (_RB, _N), 1)
    bits = lax.bitcast_convert_type(jnp.maximum(d, 0.0), jnp.int32)
    key = (bits & jnp.int32(-2048)) | iota

    riota = lax.broadcasted_iota(jnp.int32, (_RB, _N), 0)
    key = jnp.where(iota == riota + rb * _RB, jnp.int32(0x7FFFFFFF), key)

    for t in range(_K):
        mkey = jnp.min(key, axis=1, keepdims=True)
        cmp = key == mkey
        dsel = jnp.min(jnp.where(cmp, d, 3.0e38), axis=1,
                       keepdims=True)
        lbl_j = jnp.max(jnp.where(cmp, lbl_row, -1), axis=1,
                        keepdims=True)
        key = jnp.where(cmp, jnp.int32(0x7FFFFFFF), key)
        sign = jnp.where(lbl_j == lbl_col, 1.0, -1.0).astype(jnp.float32)
        idx_ref[:, t:t + 1] = (mkey & jnp.int32(2047)) + b * _N
        w_ref[:, t:t + 1] = sign * jnp.exp(-0.5 * dsel)


def _pair_loss_kernel(t_hbm, idx_hbm, w_hbm, out_hbm,
                      idx_a, w_a, tj_a, ti_a, idx_b, w_b, tj_b, ti_b,
                      acc_v, sem_a, sem_b):
    info = plsc.get_sparse_core_info()
    nc = info.num_cores
    wid = lax.axis_index("s") * nc + lax.axis_index("c")

    def stage(c, idx_v, w_v, tj_v, ti_v, sem):
        base_r = wid * _RPW + c * _CH
        base_p = base_r * _K
        pltpu.sync_copy(idx_hbm.at[pl.ds(base_p, _CH * _K)], idx_v)
        pltpu.sync_copy(w_hbm.at[pl.ds(base_p, _CH * _K)], w_v)
        pltpu.sync_copy(t_hbm.at[pl.ds(base_r, _CH)], ti_v)
        return pltpu.async_copy(t_hbm.at[idx_v], tj_v, sem)

    def accum(w_v, tj_v, ti_v, acc):
        wv = [w_v[pl.ds(0, 16)], w_v[pl.ds(16, 16)],
              w_v[pl.ds(32, 16)], w_v[pl.ds(40, 16)]]

        def wget(p):
            return wv[3][p - 40] if p >= 48 else wv[p // 16][p % 16]

        for row in range(_CH):
            ti = [ti_v[row, pl.ds(j * 16, 16)] for j in range(_NV)]
            for k in range(_K):
                pair = row * _K + k
                pacc = jnp.zeros((16,), jnp.float32)
                for j in range(_NV):
                    diff = ti[j] - tj_v[pair, pl.ds(j * 16, 16)]
                    pacc += diff * diff
                acc += wget(pair) * pacc
        return acc

    def pair_body(t, acc):
        cp_a = stage(2 * t, idx_a, w_a, tj_a, ti_a, sem_a)
        cp_b = stage(2 * t + 1, idx_b, w_b, tj_b, ti_b, sem_b)
        cp_a.wait()
        acc = accum(w_a, tj_a, ti_a, acc)
        cp_b.wait()
        acc = accum(w_b, tj_b, ti_b, acc)
        return acc

    acc = lax.fori_loop(0, _NCH // 2, pair_body, jnp.zeros((16,), jnp.float32))
    acc_v[...] = acc
    pltpu.sync_copy(acc_v, out_hbm.at[wid])


def kernel(logits, labels, ins_T):
    x = logits

    idx, wsel = pl.pallas_call(
        _topk_kernel,
        grid=(_B, _NRB),
        in_specs=[
            pl.BlockSpec((1, _C, _RB), lambda b, rb: (b, 0, rb)),
            pl.BlockSpec((1, _C, _N), lambda b, rb: (b, 0, 0)),
            pl.BlockSpec((1, 1, _N), lambda b, rb: (b, 0, 0)),
            pl.BlockSpec((1, _RB, 1), lambda b, rb: (b, rb, 0)),
        ],
        out_specs=[
            pl.BlockSpec((_RB, _K), lambda b, rb: (b * _NRB + rb, 0)),
            pl.BlockSpec((_RB, _K), lambda b, rb: (b * _NRB + rb, 0)),
        ],
        out_shape=[
            jax.ShapeDtypeStruct((_B * _N, _K), jnp.int32),
            jax.ShapeDtypeStruct((_B * _N, _K), jnp.float32),
        ],
    )(x, x, labels.reshape(_B, 1, _N), labels.reshape(_B, _N, 1))

    t_pad = jnp.pad(ins_T.reshape(_B * _N, _D_T),
                    ((0, 0), (0, _D_PAD - _D_T)))
    mesh = plsc.VectorSubcoreMesh(core_axis_name="c", subcore_axis_name="s")

    sc = functools.partial(
        pl.kernel, mesh=mesh,
        out_type=jax.ShapeDtypeStruct((_NW, 16), jnp.float32),
        scratch_types=[
            pltpu.VMEM((_CH * _K,), jnp.int32),
            pltpu.VMEM((_CH * _K,), jnp.float32),
            pltpu.VMEM((_CH * _K, _D_PAD), jnp.float32),
            pltpu.VMEM((_CH, _D_PAD), jnp.float32),
            pltpu.VMEM((_CH * _K,), jnp.int32),
            pltpu.VMEM((_CH * _K,), jnp.float32),
            pltpu.VMEM((_CH * _K, _D_PAD), jnp.float32),
            pltpu.VMEM((_CH, _D_PAD), jnp.float32),
            pltpu.VMEM((16,), jnp.float32),
            pltpu.SemaphoreType.DMA,
            pltpu.SemaphoreType.DMA,
        ],
    )(_pair_loss_kernel)

    partials = sc(t_pad, idx.reshape(-1), wsel.reshape(-1))

    return jnp.sum(partials) / jnp.float32(_B * _N * _K)


# speedup vs baseline: 1.4020x; 1.3688x over previous
import functools

import jax
import jax.numpy as jnp
from jax import lax
from jax.experimental import pallas as pl
from jax.experimental.pallas import tpu as pltpu
from jax.experimental.pallas import tpu_sc as plsc

_B, _C, _N = 8, 17, 2048
_K = 7
_RB = 512
_NRB = _N // _RB
_D_T = 289
_D_PAD = 384
_NV = 19

_NW = 32
_RPW = (_B * _N) // _NW
_CH = 8
_NCH = _RPW // _CH


def _dot3(a, b, dims):
    a_hi = a.astype(jnp.bfloat16).astype(jnp.float32)
    a_lo = a - a_hi
    b_hi = b.astype(jnp.bfloat16).astype(jnp.float32)
    b_lo = b - b_hi
    dd = (dims, ((), ()))
    out = lax.dot_general(a_hi, b_hi, dd, preferred_element_type=jnp.float32)
    out += lax.dot_general(a_hi, b_lo, dd, preferred_element_type=jnp.float32)
    out += lax.dot_general(a_lo, b_hi, dd, preferred_element_type=jnp.float32)
    return out


def _topk_kernel(x_rows_ref, x_full_ref, lblr_ref, lblc_ref, idx_ref, w_ref):
    b = pl.program_id(0)
    rb = pl.program_id(1)

    xr = x_rows_ref[0]
    xf = x_full_ref[0]
    ones_c = jnp.ones((_C, 1), dtype=jnp.float32)

    s_col = lax.dot_general(xr * xr, ones_c, (((0,), (0,)), ((), ())),
                            preferred_element_type=jnp.float32)
    s_row = lax.dot_general(ones_c, xf * xf, (((0,), (0,)), ((), ())),
                            preferred_element_type=jnp.float32)
    g = _dot3(xr, xf, ((0,), (0,)))
    d = s_col + s_row - 2.0 * g

    lbl_row = lblr_ref[0]
    lbl_col = lblc_ref[0]

    iota = lax.broadcasted_iota(jnp.int32, (_RB, _N), 1)
    dc = jnp.maximum(d, 0.0)
    bits = lax.bitcast_convert_type(dc, jnp.int32)
    key = (bits & jnp.int32(-2048)) | iota

    riota = lax.broadcasted_iota(jnp.int32, (_RB, _N), 0)
    key = jnp.where(iota == riota + rb * _RB, jnp.int32(0x7FFFFFFF), key)

    ds = jnp.where(lbl_row == lbl_col, dc, -dc)

    for t in range(_K):
        mkey = jnp.min(key, axis=1, keepdims=True)
        cmp = key == mkey
        dssel = jnp.sum(jnp.where(cmp, ds, 0.0), axis=1,
                        keepdims=True)
        key = jnp.where(cmp, jnp.int32(0x7FFFFFFF), key)
        sign = jnp.where(dssel >= 0.0, 1.0, -1.0).astype(jnp.float32)
        idx_ref[:, t:t + 1] = (mkey & jnp.int32(2047)) + b * _N
        w_ref[:, t:t + 1] = sign * jnp.exp(-0.5 * jnp.abs(dssel))


def _pair_loss_kernel(t_hbm, idx_hbm, w_hbm, out_hbm,
                      idx_a, w_a, tj_a, ti_a, idx_b, w_b, tj_b, ti_b,
                      acc_v, sem_a, sem_b):
    info = plsc.get_sparse_core_info()
    nc = info.num_cores
    wid = lax.axis_index("s") * nc + lax.axis_index("c")

    def stage(c, idx_v, w_v, tj_v, ti_v, sem):
        base_r = wid * _RPW + c * _CH
        base_p = base_r * _K
        pltpu.sync_copy(idx_hbm.at[pl.ds(base_p, _CH * _K)], idx_v)
        pltpu.sync_copy(w_hbm.at[pl.ds(base_p, _CH * _K)], w_v)
        pltpu.sync_copy(t_hbm.at[pl.ds(base_r, _CH)], ti_v)
        return pltpu.async_copy(t_hbm.at[idx_v], tj_v, sem)

    def accum(w_v, tj_v, ti_v, acc):
        wv = [w_v[pl.ds(0, 16)], w_v[pl.ds(16, 16)],
              w_v[pl.ds(32, 16)], w_v[pl.ds(40, 16)]]

        def wget(p):
            return wv[3][p - 40] if p >= 48 else wv[p // 16][p % 16]

        for row in range(_CH):
            ti = [ti_v[row, pl.ds(j * 16, 16)] for j in range(_NV)]
            for k in range(_K):
                pair = row * _K + k
                pacc = jnp.zeros((16,), jnp.float32)
                for j in range(_NV):
                    diff = ti[j] - tj_v[pair, pl.ds(j * 16, 16)]
                    pacc += diff * diff
                acc += wget(pair) * pacc
        return acc

    def pair_body(t, acc):
        cp_a = stage(2 * t, idx_a, w_a, tj_a, ti_a, sem_a)
        cp_b = stage(2 * t + 1, idx_b, w_b, tj_b, ti_b, sem_b)
        cp_a.wait()
        acc = accum(w_a, tj_a, ti_a, acc)
        cp_b.wait()
        acc = accum(w_b, tj_b, ti_b, acc)
        return acc

    acc = lax.fori_loop(0, _NCH // 2, pair_body, jnp.zeros((16,), jnp.float32))
    acc_v[...] = acc
    pltpu.sync_copy(acc_v, out_hbm.at[wid])


def kernel(logits, labels, ins_T):
    x = logits

    idx, wsel = pl.pallas_call(
        _topk_kernel,
        grid=(_B, _NRB),
        in_specs=[
            pl.BlockSpec((1, _C, _RB), lambda b, rb: (b, 0, rb)),
            pl.BlockSpec((1, _C, _N), lambda b, rb: (b, 0, 0)),
            pl.BlockSpec((1, 1, _N), lambda b, rb: (b, 0, 0)),
            pl.BlockSpec((1, _RB, 1), lambda b, rb: (b, rb, 0)),
        ],
        out_specs=[
            pl.BlockSpec((_RB, _K), lambda b, rb: (b * _NRB + rb, 0)),
            pl.BlockSpec((_RB, _K), lambda b, rb: (b * _NRB + rb, 0)),
        ],
        out_shape=[
            jax.ShapeDtypeStruct((_B * _N, _K), jnp.int32),
            jax.ShapeDtypeStruct((_B * _N, _K), jnp.float32),
        ],
    )(x, x, labels.reshape(_B, 1, _N), labels.reshape(_B, _N, 1))

    t_pad = jnp.pad(ins_T.reshape(_B * _N, _D_T),
                    ((0, 0), (0, _D_PAD - _D_T)))
    mesh = plsc.VectorSubcoreMesh(core_axis_name="c", subcore_axis_name="s")

    sc = functools.partial(
        pl.kernel, mesh=mesh,
        out_type=jax.ShapeDtypeStruct((_NW, 16), jnp.float32),
        scratch_types=[
            pltpu.VMEM((_CH * _K,), jnp.int32),
            pltpu.VMEM((_CH * _K,), jnp.float32),
            pltpu.VMEM((_CH * _K, _D_PAD), jnp.float32),
            pltpu.VMEM((_CH, _D_PAD), jnp.float32),
            pltpu.VMEM((_CH * _K,), jnp.int32),
            pltpu.VMEM((_CH * _K,), jnp.float32),
            pltpu.VMEM((_CH * _K, _D_PAD), jnp.float32),
            pltpu.VMEM((_CH, _D_PAD), jnp.float32),
            pltpu.VMEM((16,), jnp.float32),
            pltpu.SemaphoreType.DMA,
            pltpu.SemaphoreType.DMA,
        ],
    )(_pair_loss_kernel)

    partials = sc(t_pad, idx.reshape(-1), wsel.reshape(-1))

    return jnp.sum(partials) / jnp.float32(_B * _N * _K)


# speedup vs baseline: 1.7780x; 1.2682x over previous
import functools

import jax
import jax.numpy as jnp
from jax import lax
from jax.experimental import pallas as pl
from jax.experimental.pallas import tpu as pltpu
from jax.experimental.pallas import tpu_sc as plsc

_B, _C, _N = 8, 17, 2048
_K = 7
_RB = 512
_NRB = _N // _RB
_D_T = 289
_D_PAD = 384
_NV = 19

_NW = 32
_RPW = (_B * _N) // _NW
_CH = 8
_NCH = _RPW // _CH


def _dot3(a, b, dims):
    a_hi = a.astype(jnp.bfloat16).astype(jnp.float32)
    a_lo = a - a_hi
    b_hi = b.astype(jnp.bfloat16).astype(jnp.float32)
    b_lo = b - b_hi
    dd = (dims, ((), ()))
    out = lax.dot_general(a_hi, b_hi, dd, preferred_element_type=jnp.float32)
    out += lax.dot_general(a_hi, b_lo, dd, preferred_element_type=jnp.float32)
    out += lax.dot_general(a_lo, b_hi, dd, preferred_element_type=jnp.float32)
    return out


def _topk_kernel(boff, x_rows_ref, x_full_ref, lblr_ref, lblc_ref,
                 idx_ref, w_ref):
    b = pl.program_id(0) + boff
    rb = pl.program_id(1)

    xr = x_rows_ref[0]
    xf = x_full_ref[0]
    ones_c = jnp.ones((_C, 1), dtype=jnp.float32)

    s_col = lax.dot_general(xr * xr, ones_c, (((0,), (0,)), ((), ())),
                            preferred_element_type=jnp.float32)
    s_row = lax.dot_general(ones_c, xf * xf, (((0,), (0,)), ((), ())),
                            preferred_element_type=jnp.float32)
    g = _dot3(xr, xf, ((0,), (0,)))
    d = s_col + s_row - 2.0 * g

    lbl_row = lblr_ref[0]
    lbl_col = lblc_ref[0]

    iota = lax.broadcasted_iota(jnp.int32, (_RB, _N), 1)
    dc = jnp.maximum(d, 0.0)
    bits = lax.bitcast_convert_type(dc, jnp.int32)
    key = (bits & jnp.int32(-2048)) | iota

    riota = lax.broadcasted_iota(jnp.int32, (_RB, _N), 0)
    key = jnp.where(iota == riota + rb * _RB, jnp.int32(0x7FFFFFFF), key)

    ds = jnp.where(lbl_row == lbl_col, dc, -dc)

    for t in range(_K):
        mkey = jnp.min(key, axis=1, keepdims=True)
        cmp = key == mkey
        dssel = jnp.sum(jnp.where(cmp, ds, 0.0), axis=1,
                        keepdims=True)
        key = jnp.where(cmp, jnp.int32(0x7FFFFFFF), key)
        sign = jnp.where(dssel >= 0.0, 1.0, -1.0).astype(jnp.float32)
        idx_ref[:, t:t + 1] = (mkey & jnp.int32(2047)) + b * _N
        w_ref[:, t:t + 1] = sign * jnp.exp(-0.5 * jnp.abs(dssel))


def _pair_loss_kernel(rpw, toff, t_hbm, idx_hbm, w_hbm, out_hbm,
                      idx_a, w_a, tj_a, ti_a, idx_b, w_b, tj_b, ti_b,
                      acc_v, sem_a, sem_b):
    info = plsc.get_sparse_core_info()
    nc = info.num_cores
    wid = lax.axis_index("s") * nc + lax.axis_index("c")

    def stage(c, idx_v, w_v, tj_v, ti_v, sem):
        base_r = wid * rpw + c * _CH
        base_p = base_r * _K
        pltpu.sync_copy(idx_hbm.at[pl.ds(base_p, _CH * _K)], idx_v)
        pltpu.sync_copy(w_hbm.at[pl.ds(base_p, _CH * _K)], w_v)
        pltpu.sync_copy(t_hbm.at[pl.ds(toff + base_r, _CH)], ti_v)
        return pltpu.async_copy(t_hbm.at[idx_v], tj_v, sem)

    def accum(w_v, tj_v, ti_v, acc):
        wv = [w_v[pl.ds(0, 16)], w_v[pl.ds(16, 16)],
              w_v[pl.ds(32, 16)], w_v[pl.ds(40, 16)]]

        def wget(p):
            return wv[3][p - 40] if p >= 48 else wv[p // 16][p % 16]

        for row in range(_CH):
            ti = [ti_v[row, pl.ds(j * 16, 16)] for j in range(_NV)]
            for k in range(_K):
                pair = row * _K + k
                pacc = jnp.zeros((16,), jnp.float32)
                for j in range(_NV):
                    diff = ti[j] - tj_v[pair, pl.ds(j * 16, 16)]
                    pacc += diff * diff
                acc += wget(pair) * pacc
        return acc

    def pair_body(t, acc):
        cp_a = stage(2 * t, idx_a, w_a, tj_a, ti_a, sem_a)
        cp_b = stage(2 * t + 1, idx_b, w_b, tj_b, ti_b, sem_b)
        cp_a.wait()
        acc = accum(w_a, tj_a, ti_a, acc)
        cp_b.wait()
        acc = accum(w_b, tj_b, ti_b, acc)
        return acc

    nch = rpw // _CH
    acc = lax.fori_loop(0, nch // 2, pair_body, jnp.zeros((16,), jnp.float32))
    acc_v[...] = acc
    pltpu.sync_copy(acc_v, out_hbm.at[wid])


def kernel(logits, labels, ins_T):
    x = logits
    _BH = _B // 2
    rows_h = _BH * _N
    rpw = rows_h // _NW

    t_pad = jnp.pad(ins_T.reshape(_B * _N, _D_T),
                    ((0, 0), (0, _D_PAD - _D_T)))
    mesh = plsc.VectorSubcoreMesh(core_axis_name="c", subcore_axis_name="s")

    partial_sums = []
    for g in range(2):
        xg = x[g * _BH:(g + 1) * _BH]
        lbl = labels[g * _BH:(g + 1) * _BH]
        idx, wsel = pl.pallas_call(
            functools.partial(_topk_kernel, g * _BH),
            grid=(_BH, _NRB),
            in_specs=[
                pl.BlockSpec((1, _C, _RB), lambda b, rb: (b, 0, rb)),
                pl.BlockSpec((1, _C, _N), lambda b, rb: (b, 0, 0)),
                pl.BlockSpec((1, 1, _N), lambda b, rb: (b, 0, 0)),
                pl.BlockSpec((1, _RB, 1), lambda b, rb: (b, rb, 0)),
            ],
            out_specs=[
                pl.BlockSpec((_RB, _K), lambda b, rb: (b * _NRB + rb, 0)),
                pl.BlockSpec((_RB, _K), lambda b, rb: (b * _NRB + rb, 0)),
            ],
            out_shape=[
                jax.ShapeDtypeStruct((rows_h, _K), jnp.int32),
                jax.ShapeDtypeStruct((rows_h, _K), jnp.float32),
            ],
        )(xg, xg, lbl.reshape(_BH, 1, _N), lbl.reshape(_BH, _N, 1))

        sc = functools.partial(
            pl.kernel, mesh=mesh,
            out_type=jax.ShapeDtypeStruct((_NW, 16), jnp.float32),
            scratch_types=[
                pltpu.VMEM((_CH * _K,), jnp.int32),
                pltpu.VMEM((_CH * _K,), jnp.float32),
                pltpu.VMEM((_CH * _K, _D_PAD), jnp.float32),
                pltpu.VMEM((_CH, _D_PAD), jnp.float32),
                pltpu.VMEM((_CH * _K,), jnp.int32),
                pltpu.VMEM((_CH * _K,), jnp.float32),
                pltpu.VMEM((_CH * _K, _D_PAD), jnp.float32),
                pltpu.VMEM((_CH, _D_PAD), jnp.float32),
                pltpu.VMEM((16,), jnp.float32),
                pltpu.SemaphoreType.DMA,
                pltpu.SemaphoreType.DMA,
            ],
        )(functools.partial(_pair_loss_kernel, rpw, g * rows_h))

        partial_sums.append(sc(t_pad, idx.reshape(-1), wsel.reshape(-1)))

    total = sum(jnp.sum(p) for p in partial_sums)
    return total / jnp.float32(_B * _N * _K)


# speedup vs baseline: 1.9968x; 1.1231x over previous
import functools

import jax
import jax.numpy as jnp
from jax import lax
from jax.experimental import pallas as pl
from jax.experimental.pallas import tpu as pltpu
from jax.experimental.pallas import tpu_sc as plsc

_B, _C, _N = 8, 17, 2048
_K = 7
_RB = 512
_NRB = _N // _RB
_D_T = 289
_D_PAD = 384
_NV = 19

_NW = 32
_RPW = (_B * _N) // _NW
_CH = 8
_NCH = _RPW // _CH


def _dot3(a, b, dims):
    a_hi = a.astype(jnp.bfloat16).astype(jnp.float32)
    a_lo = a - a_hi
    b_hi = b.astype(jnp.bfloat16).astype(jnp.float32)
    b_lo = b - b_hi
    dd = (dims, ((), ()))
    out = lax.dot_general(a_hi, b_hi, dd, preferred_element_type=jnp.float32)
    out += lax.dot_general(a_hi, b_lo, dd, preferred_element_type=jnp.float32)
    out += lax.dot_general(a_lo, b_hi, dd, preferred_element_type=jnp.float32)
    return out


def _topk_kernel(boff, x_rows_ref, x_full_ref, lblr_ref, lblc_ref,
                 idx_ref, w_ref):
    b = pl.program_id(0) + boff
    rb = pl.program_id(1)

    xr = x_rows_ref[0]
    xf = x_full_ref[0]
    ones_c = jnp.ones((_C, 1), dtype=jnp.float32)

    s_col = lax.dot_general(xr * xr, ones_c, (((0,), (0,)), ((), ())),
                            preferred_element_type=jnp.float32)
    s_row = lax.dot_general(ones_c, xf * xf, (((0,), (0,)), ((), ())),
                            preferred_element_type=jnp.float32)
    g = _dot3(xr, xf, ((0,), (0,)))
    d = s_col + s_row - 2.0 * g

    lbl_row = lblr_ref[0]
    lbl_col = lblc_ref[0]

    iota = lax.broadcasted_iota(jnp.int32, (_RB, _N), 1)
    dc = jnp.maximum(d, 0.0)
    bits = lax.bitcast_convert_type(dc, jnp.int32)
    key = (bits & jnp.int32(-2048)) | iota

    riota = lax.broadcasted_iota(jnp.int32, (_RB, _N), 0)
    key = jnp.where(iota == riota + rb * _RB, jnp.int32(0x7FFFFFFF), key)

    ds = jnp.where(lbl_row == lbl_col, dc, -dc)

    for t in range(_K):
        mkey = jnp.min(key, axis=1, keepdims=True)
        cmp = key == mkey
        dssel = jnp.sum(jnp.where(cmp, ds, 0.0), axis=1,
                        keepdims=True)
        key = jnp.where(cmp, jnp.int32(0x7FFFFFFF), key)
        sign = jnp.where(dssel >= 0.0, 1.0, -1.0).astype(jnp.float32)
        idx_ref[:, t:t + 1] = (mkey & jnp.int32(2047)) + b * _N
        w_ref[:, t:t + 1] = sign * jnp.exp(-0.5 * jnp.abs(dssel))


def _pair_loss_kernel(rpw, toff, t_hbm, idx_hbm, w_hbm, out_hbm,
                      idx_a, w_a, tj_a, ti_a, idx_b, w_b, tj_b, ti_b,
                      acc_v, sem_a, sem_b):
    info = plsc.get_sparse_core_info()
    nc = info.num_cores
    wid = lax.axis_index("s") * nc + lax.axis_index("c")

    def stage(c, idx_v, w_v, tj_v, ti_v, sem):
        base_r = wid * rpw + c * _CH
        base_p = base_r * _K
        pltpu.sync_copy(idx_hbm.at[pl.ds(base_p, _CH * _K)], idx_v)
        pltpu.sync_copy(w_hbm.at[pl.ds(base_p, _CH * _K)], w_v)
        pltpu.sync_copy(t_hbm.at[pl.ds(toff + base_r, _CH)], ti_v)
        return pltpu.async_copy(t_hbm.at[idx_v], tj_v, sem)

    def accum(w_v, tj_v, ti_v, acc):
        wv = [w_v[pl.ds(0, 16)], w_v[pl.ds(16, 16)],
              w_v[pl.ds(32, 16)], w_v[pl.ds(40, 16)]]

        def wget(p):
            return wv[3][p - 40] if p >= 48 else wv[p // 16][p % 16]

        for row in range(_CH):
            ti = [ti_v[row, pl.ds(j * 16, 16)] for j in range(_NV)]
            for k in range(_K):
                pair = row * _K + k
                pacc = jnp.zeros((16,), jnp.float32)
                for j in range(_NV):
                    diff = ti[j] - tj_v[pair, pl.ds(j * 16, 16)]
                    pacc += diff * diff
                acc += wget(pair) * pacc
        return acc

    def pair_body(t, acc):
        cp_a = stage(2 * t, idx_a, w_a, tj_a, ti_a, sem_a)
        cp_b = stage(2 * t + 1, idx_b, w_b, tj_b, ti_b, sem_b)
        cp_a.wait()
        acc = accum(w_a, tj_a, ti_a, acc)
        cp_b.wait()
        acc = accum(w_b, tj_b, ti_b, acc)
        return acc

    nch = rpw // _CH
    acc = lax.fori_loop(0, nch // 2, pair_body, jnp.zeros((16,), jnp.float32))
    acc_v[...] = acc
    pltpu.sync_copy(acc_v, out_hbm.at[wid])


def kernel(logits, labels, ins_T):
    x = logits
    _NG = 4
    _BH = _B // _NG
    rows_h = _BH * _N
    rpw = rows_h // _NW

    t_pad = jnp.pad(ins_T.reshape(_B * _N, _D_T),
                    ((0, 0), (0, _D_PAD - _D_T)))
    mesh = plsc.VectorSubcoreMesh(core_axis_name="c", subcore_axis_name="s")

    partial_sums = []
    for g in range(_NG):
        xg = x[g * _BH:(g + 1) * _BH]
        lbl = labels[g * _BH:(g + 1) * _BH]
        idx, wsel = pl.pallas_call(
            functools.partial(_topk_kernel, g * _BH),
            grid=(_BH, _NRB),
            in_specs=[
                pl.BlockSpec((1, _C, _RB), lambda b, rb: (b, 0, rb)),
                pl.BlockSpec((1, _C, _N), lambda b, rb: (b, 0, 0)),
                pl.BlockSpec((1, 1, _N), lambda b, rb: (b, 0, 0)),
                pl.BlockSpec((1, _RB, 1), lambda b, rb: (b, rb, 0)),
            ],
            out_specs=[
                pl.BlockSpec((_RB, _K), lambda b, rb: (b * _NRB + rb, 0)),
                pl.BlockSpec((_RB, _K), lambda b, rb: (b * _NRB + rb, 0)),
            ],
            out_shape=[
                jax.ShapeDtypeStruct((rows_h, _K), jnp.int32),
                jax.ShapeDtypeStruct((rows_h, _K), jnp.float32),
            ],
        )(xg, xg, lbl.reshape(_BH, 1, _N), lbl.reshape(_BH, _N, 1))

        sc = functools.partial(
            pl.kernel, mesh=mesh,
            out_type=jax.ShapeDtypeStruct((_NW, 16), jnp.float32),
            scratch_types=[
                pltpu.VMEM((_CH * _K,), jnp.int32),
                pltpu.VMEM((_CH * _K,), jnp.float32),
                pltpu.VMEM((_CH * _K, _D_PAD), jnp.float32),
                pltpu.VMEM((_CH, _D_PAD), jnp.float32),
                pltpu.VMEM((_CH * _K,), jnp.int32),
                pltpu.VMEM((_CH * _K,), jnp.float32),
                pltpu.VMEM((_CH * _K, _D_PAD), jnp.float32),
                pltpu.VMEM((_CH, _D_PAD), jnp.float32),
                pltpu.VMEM((16,), jnp.float32),
                pltpu.SemaphoreType.DMA,
                pltpu.SemaphoreType.DMA,
            ],
        )(functools.partial(_pair_loss_kernel, rpw, g * rows_h))

        partial_sums.append(sc(t_pad, idx.reshape(-1), wsel.reshape(-1)))

    total = sum(jnp.sum(p) for p in partial_sums)
    return total / jnp.float32(_B * _N * _K)
